# Initial kernel scaffold; baseline (speedup 1.0000x reference)
import jax
import jax.numpy as jnp
from jax.experimental import pallas as pl


def kernel(pos, start_idx):
    raise NotImplementedError("write your pallas kernel here")


# speedup vs baseline: 12.1600x; 12.1600x over previous
import jax
import jax.numpy as jnp
import numpy as np
from jax import lax
from jax.experimental import pallas as pl
from jax.experimental.pallas import tpu as pltpu
from jax.experimental.pallas import tpu_sc as plsc

B = 16
N = 8192
NPTS = 2048
L = 16
CHUNKS = N // L

_NEG = np.float32(-np.inf)
_INF = np.float32(np.inf)
_BIGI = np.int32(2**31 - 1)


def _fps_body(pos_hbm, start_hbm, out_hbm, x_v, y_v, z_v, dist_v, res_v,
              start_v, sem):
    c = lax.axis_index("c")
    s = lax.axis_index("s")
    b = c * 16 + s

    @pl.when(b < B)
    def _():
        pltpu.sync_copy(pos_hbm.at[b, 0], x_v)
        pltpu.sync_copy(pos_hbm.at[b, 1], y_v)
        pltpu.sync_copy(pos_hbm.at[b, 2], z_v)
        pltpu.sync_copy(start_hbm, start_v)

        lanes = lax.iota(jnp.int32, L)
        lane0 = lanes == 0
        fv0 = start_v[...]

        plsc.store_scatter(res_v, [jnp.zeros((L,), jnp.int32)], fv0,
                           mask=lane0)

        def initc(j, carry):
            dist_v[pl.ds(j * L, L)] = jnp.full((L,), _INF)
            return carry

        lax.fori_loop(0, CHUNKS, initc, 0)

        def step(i, fv):
            cx = plsc.load_gather(x_v, [fv])
            cy = plsc.load_gather(y_v, [fv])
            cz = plsc.load_gather(z_v, [fv])

            def chunk(j, carry):
                vmax, vidx, base = carry
                sl = pl.ds(j * L, L)
                dx = x_v[sl] - cx
                dy = y_v[sl] - cy
                dz = z_v[sl] - cz
                d = dx * dx + dy * dy + dz * dz
                dm = jnp.minimum(dist_v[sl], d)
                dist_v[sl] = dm
                m = dm > vmax
                vmax = jnp.where(m, dm, vmax)
                vidx = jnp.where(m, base, vidx)
                return vmax, vidx, base + L

            vmax, vidx, _ = lax.fori_loop(
                0, CHUNKS, chunk,
                (jnp.full((L,), _NEG), jnp.full((L,), _BIGI), lanes),
            )
            gmax = jnp.max(vmax)
            nxt = jnp.min(jnp.where(vmax == gmax, vidx, _BIGI))
            nv = jnp.full((L,), nxt)
            plsc.store_scatter(res_v, [jnp.full((L,), i + 1)], nv, mask=lane0)
            return nv

        lax.fori_loop(0, NPTS - 1, step, fv0)
        pltpu.sync_copy(res_v, out_hbm.at[b])


@jax.jit
def _fps_call(pos_t, start):
    mesh = plsc.VectorSubcoreMesh(
        core_axis_name="c", subcore_axis_name="s", num_cores=2, num_subcores=16
    )
    return pl.kernel(
        _fps_body,
        out_type=jax.ShapeDtypeStruct((B, NPTS), jnp.int32),
        mesh=mesh,
        compiler_params=pltpu.CompilerParams(
            use_tc_tiling_on_sc=False, needs_layout_passes=False
        ),
        scratch_types=[
            pltpu.VMEM((N,), jnp.float32),
            pltpu.VMEM((N,), jnp.float32),
            pltpu.VMEM((N,), jnp.float32),
            pltpu.VMEM((N,), jnp.float32),
            pltpu.VMEM((NPTS,), jnp.int32),
            pltpu.VMEM((L,), jnp.int32),
            pltpu.SemaphoreType.DMA,
        ],
    )(pos_t, start)


def kernel(pos, start_idx):
    pos_t = jnp.transpose(pos, (0, 2, 1))
    start = jnp.full((L,), start_idx, dtype=jnp.int32)
    return _fps_call(pos_t, start)


# speedup vs baseline: 20.1883x; 1.6602x over previous
import jax
import jax.numpy as jnp
import numpy as np
from jax import lax
from jax.experimental import pallas as pl
from jax.experimental.pallas import tpu as pltpu
from jax.experimental.pallas import tpu_sc as plsc

B = 16
N = 8192
NPTS = 2048
L = 16
HALF = N // 2
CHUNKS = HALF // L

_NEG = np.float32(-np.inf)
_INF = np.float32(np.inf)
_BIGI = np.int32(2**31 - 1)


def _fps_body(pos_hbm, start_hbm, out_hbm, x_v, y_v, z_v, xh_v, yh_v, zh_v,
              dist_v, res_v, start_v, rec_v, prec_v, shared, sem):
    c = lax.axis_index("c")
    s = lax.axis_index("s")
    b = c * 8 + lax.shift_right_logical(s, 1)
    h = lax.bitwise_and(s, 1)
    off = h * HALF

    pltpu.sync_copy(pos_hbm.at[b, 0], x_v)
    pltpu.sync_copy(pos_hbm.at[b, 1], y_v)
    pltpu.sync_copy(pos_hbm.at[b, 2], z_v)
    pltpu.sync_copy(pos_hbm.at[b, 0, pl.ds(off, HALF)], xh_v)
    pltpu.sync_copy(pos_hbm.at[b, 1, pl.ds(off, HALF)], yh_v)
    pltpu.sync_copy(pos_hbm.at[b, 2, pl.ds(off, HALF)], zh_v)
    pltpu.sync_copy(start_hbm, start_v)

    lanes = lax.iota(jnp.int32, L)
    lane0 = lanes == 0
    zeros = jnp.zeros((L,), jnp.int32)
    ones = jnp.full((L,), 1, jnp.int32)
    fv0 = start_v[...]

    plsc.store_scatter(res_v, [zeros], fv0, mask=lane0)

    def initc(j, carry):
        dist_v[pl.ds(j * L, L)] = jnp.full((L,), _INF)
        return carry

    lax.fori_loop(0, CHUNKS, initc, 0)

    def step(i, _):
        fv = plsc.load_gather(res_v, [jnp.full((L,), i)])
        cx = plsc.load_gather(x_v, [fv])
        cy = plsc.load_gather(y_v, [fv])
        cz = plsc.load_gather(z_v, [fv])

        def chunk(j, carry):
            vmax, vidx, base = carry
            sl = pl.ds(j * L, L)
            dx = xh_v[sl] - cx
            dy = yh_v[sl] - cy
            dz = zh_v[sl] - cz
            d = dx * dx + dy * dy + dz * dz
            dm = jnp.minimum(dist_v[sl], d)
            dist_v[sl] = dm
            m = dm > vmax
            vmax = jnp.where(m, dm, vmax)
            vidx = jnp.where(m, base, vidx)
            return vmax, vidx, base + L

        vmax, vidx, _ = lax.fori_loop(
            0, CHUNKS, chunk,
            (jnp.full((L,), _NEG), jnp.full((L,), _BIGI), off + lanes),
        )
        gmax = jnp.max(vmax)
        myidx = jnp.min(jnp.where(vmax == gmax, vidx, _BIGI))
        gmax_v = jnp.full((L,), gmax)
        myidx_v = jnp.full((L,), myidx)

        p = lax.bitwise_and(i, 1)
        rec_v[...] = jnp.where(lane0, gmax_v, plsc.bitcast(myidx_v, jnp.float32))
        pltpu.sync_copy(rec_v, shared.at[s, p])
        plsc.subcore_barrier()
        pltpu.sync_copy(shared.at[lax.bitwise_xor(s, 1), p], prec_v)

        omax_v = plsc.load_gather(prec_v, [zeros])
        oidx_v = plsc.bitcast(plsc.load_gather(prec_v, [ones]), jnp.int32)
        better = (omax_v > gmax_v) | ((omax_v == gmax_v) & (oidx_v < myidx_v))
        nxt_v = jnp.where(better, oidx_v, myidx_v)

        plsc.store_scatter(res_v, [jnp.full((L,), i + 1)], nxt_v, mask=lane0)
        return 0

    lax.fori_loop(0, NPTS - 1, step, 0)

    @pl.when(h == 0)
    def _():
        pltpu.sync_copy(res_v, out_hbm.at[b])


@jax.jit
def _fps_call(pos_t, start):
    mesh = plsc.VectorSubcoreMesh(
        core_axis_name="c", subcore_axis_name="s", num_cores=2, num_subcores=16
    )
    return pl.kernel(
        _fps_body,
        out_type=jax.ShapeDtypeStruct((B, NPTS), jnp.int32),
        mesh=mesh,
        compiler_params=pltpu.CompilerParams(
            use_tc_tiling_on_sc=False, needs_layout_passes=False
        ),
        scratch_types=[
            pltpu.VMEM((N,), jnp.float32),
            pltpu.VMEM((N,), jnp.float32),
            pltpu.VMEM((N,), jnp.float32),
            pltpu.VMEM((HALF,), jnp.float32),
            pltpu.VMEM((HALF,), jnp.float32),
            pltpu.VMEM((HALF,), jnp.float32),
            pltpu.VMEM((HALF,), jnp.float32),
            pltpu.VMEM((NPTS,), jnp.int32),
            pltpu.VMEM((L,), jnp.int32),
            pltpu.VMEM((L,), jnp.float32),
            pltpu.VMEM((L,), jnp.float32),
            pltpu.VMEM_SHARED((16, 2, L), jnp.float32),
            pltpu.SemaphoreType.DMA,
        ],
    )(pos_t, start)


def kernel(pos, start_idx):
    pos_t = jnp.transpose(pos, (0, 2, 1))
    start = jnp.full((L,), start_idx, dtype=jnp.int32)
    return _fps_call(pos_t, start)


# speedup vs baseline: 28.4106x; 1.4073x over previous
import jax
import jax.numpy as jnp
import numpy as np
from jax import lax
from jax.experimental import pallas as pl
from jax.experimental.pallas import tpu as pltpu
from jax.experimental.pallas import tpu_sc as plsc

B = 16
N = 8192
NPTS = 2048
L = 16
HALF = N // 2
CHUNKS = HALF // L

_NEG = np.float32(-np.inf)
_INF = np.float32(np.inf)
_BIGI = np.int32(2**31 - 1)


def _fps_body(pos_hbm, start_hbm, out_hbm, x_v, y_v, z_v, xh_v, yh_v, zh_v,
              dist_v, res_v, start_v, rec_v, prec_v, shared, sem):
    c = lax.axis_index("c")
    s = lax.axis_index("s")
    b = c * 8 + lax.shift_right_logical(s, 1)
    h = lax.bitwise_and(s, 1)
    off = h * HALF

    pltpu.sync_copy(pos_hbm.at[b, 0], x_v)
    pltpu.sync_copy(pos_hbm.at[b, 1], y_v)
    pltpu.sync_copy(pos_hbm.at[b, 2], z_v)
    pltpu.sync_copy(pos_hbm.at[b, 0, pl.ds(off, HALF)], xh_v)
    pltpu.sync_copy(pos_hbm.at[b, 1, pl.ds(off, HALF)], yh_v)
    pltpu.sync_copy(pos_hbm.at[b, 2, pl.ds(off, HALF)], zh_v)
    pltpu.sync_copy(start_hbm, start_v)

    lanes = lax.iota(jnp.int32, L)
    lane0 = lanes == 0
    zeros = jnp.zeros((L,), jnp.int32)
    ones = jnp.full((L,), 1, jnp.int32)
    fv0 = start_v[...]

    plsc.store_scatter(res_v, [zeros], fv0, mask=lane0)

    def initc(j, carry):
        dist_v[pl.ds(j * L, L)] = jnp.full((L,), _INF)
        return carry

    lax.fori_loop(0, CHUNKS, initc, 0)

    def step(i, _):
        fv = plsc.load_gather(res_v, [jnp.full((L,), i)])
        cx = plsc.load_gather(x_v, [fv])
        cy = plsc.load_gather(y_v, [fv])
        cz = plsc.load_gather(z_v, [fv])

        def chunk(j, carry):
            vmax, vidx, base = carry
            s0 = pl.ds(j * 2 * L, L)
            s1 = pl.ds(j * 2 * L + L, L)
            dx0 = xh_v[s0] - cx
            dx1 = xh_v[s1] - cx
            dy0 = yh_v[s0] - cy
            dy1 = yh_v[s1] - cy
            dz0 = zh_v[s0] - cz
            dz1 = zh_v[s1] - cz
            d0 = dx0 * dx0 + dy0 * dy0 + dz0 * dz0
            d1 = dx1 * dx1 + dy1 * dy1 + dz1 * dz1
            dm0 = jnp.minimum(dist_v[s0], d0)
            dm1 = jnp.minimum(dist_v[s1], d1)
            dist_v[s0] = dm0
            dist_v[s1] = dm1
            m0 = dm0 > vmax
            vmax = jnp.where(m0, dm0, vmax)
            vidx = jnp.where(m0, base, vidx)
            m1 = dm1 > vmax
            vmax = jnp.where(m1, dm1, vmax)
            vidx = jnp.where(m1, base + L, vidx)
            return vmax, vidx, base + 2 * L

        vmax, vidx, _ = lax.fori_loop(
            0, CHUNKS // 2, chunk,
            (jnp.full((L,), _NEG), jnp.full((L,), _BIGI), off + lanes),
        )
        gmax = jnp.max(vmax)
        myidx = jnp.min(jnp.where(vmax == gmax, vidx, _BIGI))
        gmax_v = jnp.full((L,), gmax)
        myidx_v = jnp.full((L,), myidx)

        p = lax.bitwise_and(i, 1)
        rec_v[...] = jnp.where(lane0, gmax_v, plsc.bitcast(myidx_v, jnp.float32))
        pltpu.sync_copy(rec_v, shared.at[s, p])
        plsc.subcore_barrier()
        pltpu.sync_copy(shared.at[lax.bitwise_xor(s, 1), p], prec_v)

        omax_v = plsc.load_gather(prec_v, [zeros])
        oidx_v = plsc.bitcast(plsc.load_gather(prec_v, [ones]), jnp.int32)
        better = (omax_v > gmax_v) | ((omax_v == gmax_v) & (oidx_v < myidx_v))
        nxt_v = jnp.where(better, oidx_v, myidx_v)

        plsc.store_scatter(res_v, [jnp.full((L,), i + 1)], nxt_v, mask=lane0)
        return 0

    lax.fori_loop(0, NPTS - 1, step, 0)

    @pl.when(h == 0)
    def _():
        pltpu.sync_copy(res_v, out_hbm.at[b])


@jax.jit
def _fps_call(pos_t, start):
    mesh = plsc.VectorSubcoreMesh(
        core_axis_name="c", subcore_axis_name="s", num_cores=2, num_subcores=16
    )
    return pl.kernel(
        _fps_body,
        out_type=jax.ShapeDtypeStruct((B, NPTS), jnp.int32),
        mesh=mesh,
        compiler_params=pltpu.CompilerParams(
            use_tc_tiling_on_sc=False, needs_layout_passes=False
        ),
        scratch_types=[
            pltpu.VMEM((N,), jnp.float32),
            pltpu.VMEM((N,), jnp.float32),
            pltpu.VMEM((N,), jnp.float32),
            pltpu.VMEM((HALF,), jnp.float32),
            pltpu.VMEM((HALF,), jnp.float32),
            pltpu.VMEM((HALF,), jnp.float32),
            pltpu.VMEM((HALF,), jnp.float32),
            pltpu.VMEM((NPTS,), jnp.int32),
            pltpu.VMEM((L,), jnp.int32),
            pltpu.VMEM((L,), jnp.float32),
            pltpu.VMEM((L,), jnp.float32),
            pltpu.VMEM_SHARED((16, 2, L), jnp.float32),
            pltpu.SemaphoreType.DMA,
        ],
    )(pos_t, start)


def kernel(pos, start_idx):
    pos_t = jnp.transpose(pos, (0, 2, 1))
    start = jnp.full((L,), start_idx, dtype=jnp.int32)
    return _fps_call(pos_t, start)


# speedup vs baseline: 28.4278x; 1.0006x over previous
import jax
import jax.numpy as jnp
import numpy as np
from jax import lax
from jax.experimental import pallas as pl
from jax.experimental.pallas import tpu as pltpu
from jax.experimental.pallas import tpu_sc as plsc

B = 16
N = 8192
NPTS = 2048
L = 16
HALF = N // 2
CHUNKS = HALF // L

_NEG = np.float32(-np.inf)
_INF = np.float32(np.inf)
_BIGI = np.int32(2**31 - 1)


def _fps_body(pos_hbm, start_hbm, out_hbm, x_v, y_v, z_v, xh_v, yh_v, zh_v,
              dist_v, res_v, start_v, rec_v, prec_v, shared, sem):
    c = lax.axis_index("c")
    s = lax.axis_index("s")
    b = c * 8 + lax.shift_right_logical(s, 1)
    h = lax.bitwise_and(s, 1)
    off = h * HALF

    pltpu.sync_copy(pos_hbm.at[b, 0], x_v)
    pltpu.sync_copy(pos_hbm.at[b, 1], y_v)
    pltpu.sync_copy(pos_hbm.at[b, 2], z_v)
    pltpu.sync_copy(pos_hbm.at[b, 0, pl.ds(off, HALF)], xh_v)
    pltpu.sync_copy(pos_hbm.at[b, 1, pl.ds(off, HALF)], yh_v)
    pltpu.sync_copy(pos_hbm.at[b, 2, pl.ds(off, HALF)], zh_v)
    pltpu.sync_copy(start_hbm, start_v)

    lanes = lax.iota(jnp.int32, L)
    lane0 = lanes == 0
    zeros = jnp.zeros((L,), jnp.int32)
    ones = jnp.full((L,), 1, jnp.int32)
    fv0 = start_v[...]

    plsc.store_scatter(res_v, [zeros], fv0, mask=lane0)

    def initc(j, carry):
        dist_v[pl.ds(j * L, L)] = jnp.full((L,), _INF)
        return carry

    lax.fori_loop(0, CHUNKS, initc, 0)

    def step(i, _):
        fv = plsc.load_gather(res_v, [jnp.full((L,), i)])
        cx = plsc.load_gather(x_v, [fv])
        cy = plsc.load_gather(y_v, [fv])
        cz = plsc.load_gather(z_v, [fv])

        U = 4

        def chunk(j, carry):
            vmax, vidx, base = carry
            sls = [pl.ds(j * U * L + u * L, L) for u in range(U)]
            dms = []
            for sl in sls:
                dx = xh_v[sl] - cx
                dy = yh_v[sl] - cy
                dz = zh_v[sl] - cz
                d = dx * dx + dy * dy + dz * dz
                dm = jnp.minimum(dist_v[sl], d)
                dms.append(dm)
            for u, (sl, dm) in enumerate(zip(sls, dms)):
                dist_v[sl] = dm
                m = dm > vmax
                vmax = jnp.where(m, dm, vmax)
                vidx = jnp.where(m, base + u * L, vidx)
            return vmax, vidx, base + U * L

        vmax, vidx, _ = lax.fori_loop(
            0, CHUNKS // U, chunk,
            (jnp.full((L,), _NEG), jnp.full((L,), _BIGI), off + lanes),
        )
        gmax = jnp.max(vmax)
        myidx = jnp.min(jnp.where(vmax == gmax, vidx, _BIGI))
        gmax_v = jnp.full((L,), gmax)
        myidx_v = jnp.full((L,), myidx)

        p = lax.bitwise_and(i, 1)
        rec_v[...] = jnp.where(lane0, gmax_v, plsc.bitcast(myidx_v, jnp.float32))
        pltpu.sync_copy(rec_v, shared.at[s, p])
        plsc.subcore_barrier()
        pltpu.sync_copy(shared.at[lax.bitwise_xor(s, 1), p], prec_v)

        omax_v = plsc.load_gather(prec_v, [zeros])
        oidx_v = plsc.bitcast(plsc.load_gather(prec_v, [ones]), jnp.int32)
        better = (omax_v > gmax_v) | ((omax_v == gmax_v) & (oidx_v < myidx_v))
        nxt_v = jnp.where(better, oidx_v, myidx_v)

        plsc.store_scatter(res_v, [jnp.full((L,), i + 1)], nxt_v, mask=lane0)
        return 0

    lax.fori_loop(0, NPTS - 1, step, 0)

    @pl.when(h == 0)
    def _():
        pltpu.sync_copy(res_v, out_hbm.at[b])


@jax.jit
def _fps_call(pos_t, start):
    mesh = plsc.VectorSubcoreMesh(
        core_axis_name="c", subcore_axis_name="s", num_cores=2, num_subcores=16
    )
    return pl.kernel(
        _fps_body,
        out_type=jax.ShapeDtypeStruct((B, NPTS), jnp.int32),
        mesh=mesh,
        compiler_params=pltpu.CompilerParams(
            use_tc_tiling_on_sc=False, needs_layout_passes=False
        ),
        scratch_types=[
            pltpu.VMEM((N,), jnp.float32),
            pltpu.VMEM((N,), jnp.float32),
            pltpu.VMEM((N,), jnp.float32),
            pltpu.VMEM((HALF,), jnp.float32),
            pltpu.VMEM((HALF,), jnp.float32),
            pltpu.VMEM((HALF,), jnp.float32),
            pltpu.VMEM((HALF,), jnp.float32),
            pltpu.VMEM((NPTS,), jnp.int32),
            pltpu.VMEM((L,), jnp.int32),
            pltpu.VMEM((L,), jnp.float32),
            pltpu.VMEM((L,), jnp.float32),
            pltpu.VMEM_SHARED((16, 2, L), jnp.float32),
            pltpu.SemaphoreType.DMA,
        ],
    )(pos_t, start)


def kernel(pos, start_idx):
    pos_t = jnp.transpose(pos, (0, 2, 1))
    start = jnp.full((L,), start_idx, dtype=jnp.int32)
    return _fps_call(pos_t, start)
